# baseline (speedup 1.0000x reference)
import functools

import jax
import jax.numpy as jnp
from jax import lax
from jax.experimental import pallas as pl
from jax.experimental.pallas import tpu as pltpu
from jax.experimental.pallas import tpu_sc as plsc

_POS_LEN = 19
_PL1 = _POS_LEN + 1
_NUM_HIST = 5
_HW = 361
_PAD_IDX = _PL1 * _PL1 - 1
_BR = 4096
_BN = 512


def _argmax_body(x_ref, re_ref, ce_ref, comb_ref, table_ref):
    i = pl.program_id(0)

    @pl.when(i == 0)
    def _build_table():
        p = lax.broadcasted_iota(jnp.int32, (_PL1 * _PL1, _PL1), 0)
        e = lax.broadcasted_iota(jnp.int32, (_PL1 * _PL1, _PL1), 1)
        pr = (p * 410) >> 13
        oh_r = (pr == e).astype(jnp.float32)
        oh_c = (p - pr * _PL1 == e).astype(jnp.float32)
        t_r = jnp.dot(oh_r, re_ref[...], preferred_element_type=jnp.float32)
        t_c = jnp.dot(oh_c, ce_ref[...], preferred_element_type=jnp.float32)
        table_ref[...] = jnp.concatenate([t_r, t_c], axis=-1)

    flat = x_ref[...]
    s = jnp.sum(flat, axis=-1)
    m = jnp.max(flat, axis=-1)
    iota = lax.broadcasted_iota(jnp.int32, flat.shape, 1).astype(jnp.float32)
    idxf = jnp.min(jnp.where(flat == m[:, None], iota, jnp.float32(512.0)),
                   axis=-1)
    idx = idxf.astype(jnp.int32)
    has = s > 0.5
    rows = (idx * 27) >> 9
    cols = idx - rows * 19
    comb_ref[...] = jnp.where(has, rows * _PL1 + cols, _PAD_IDX)


def _extract_indices(x2d, row_embed, col_embed):
    nr = x2d.shape[0]
    return pl.pallas_call(
        _argmax_body,
        grid=(nr // _BR,),
        in_specs=[
            pl.BlockSpec((_BR, _HW), lambda i: (i, 0)),
            pl.BlockSpec((_PL1, 64), lambda i: (0, 0)),
            pl.BlockSpec((_PL1, 64), lambda i: (0, 0)),
        ],
        out_specs=[
            pl.BlockSpec((_BR,), lambda i: (i,)),
            pl.BlockSpec((_PL1 * _PL1, 128), lambda i: (0, 0)),
        ],
        out_shape=[
            jax.ShapeDtypeStruct((nr,), jnp.int32),
            jax.ShapeDtypeStruct((_PL1 * _PL1, 128), jnp.float32),
        ],
    )(x2d, row_embed, col_embed)


def _sc_gather(table, idx):
    info = plsc.get_sparse_core_info()
    nw = info.num_cores * info.num_subcores
    n = idx.shape[0]
    rpw = n // nw
    cpw = rpw // 128
    d = table.shape[1]
    mesh = plsc.VectorSubcoreMesh(core_axis_name="c", subcore_axis_name="s")

    @functools.partial(
        pl.kernel,
        mesh=mesh,
        out_type=jax.ShapeDtypeStruct((n, d), jnp.float32),
        scratch_types=[
            pltpu.VMEM((rpw,), jnp.int32),
            pltpu.VMEM((rpw, d), jnp.float32),
            pltpu.SemaphoreType.DMA,
            pltpu.SemaphoreType.DMA,
        ],
    )
    def gather_kernel(table_hbm, idx_hbm, out_hbm, idx_v, rows_v, gsem, wsem):
        wid = lax.axis_index("s") * info.num_cores + lax.axis_index("c")
        base = wid * rpw
        pltpu.sync_copy(idx_hbm.at[pl.ds(base, rpw)], idx_v)
        gathers = [
            pltpu.async_copy(table_hbm.at[idx_v.at[pl.ds(j * 128, 128)]],
                             rows_v.at[pl.ds(j * 128, 128)], gsem)
            for j in range(cpw)
        ]
        writes = []
        for j in range(cpw):
            gathers[j].wait()
            writes.append(pltpu.async_copy(
                rows_v.at[pl.ds(j * 128, 128)],
                out_hbm.at[pl.ds(base + j * 128, 128)], wsem))
        for w in writes:
            w.wait()

    return gather_kernel(table, idx)


def _mlp_body(e_ref, w1_ref, b1_ref, w2_ref, b2_ref, o_ref):
    acc = None
    for k in range(_NUM_HIST):
        part = lax.dot_general(
            e_ref[:, k, :], w1_ref[:, pl.ds(k * 128, 128)],
            (((1,), (1,)), ((), ())), preferred_element_type=jnp.float32)
        acc = part if acc is None else acc + part
    h = jnp.maximum(acc + b1_ref[...], 0.0)
    o_ref[...] = lax.dot_general(
        h, w2_ref[...], (((1,), (1,)), ((), ())),
        preferred_element_type=jnp.float32) + b2_ref[...]


def _mlp(e3, w1, b1, w2, b2):
    n = e3.shape[0]
    hidden = w1.shape[0]
    c_out = w2.shape[0]
    return pl.pallas_call(
        _mlp_body,
        grid=(n // _BN,),
        in_specs=[
            pl.BlockSpec((_BN, _NUM_HIST, 128), lambda i: (i, 0, 0)),
            pl.BlockSpec((hidden, _NUM_HIST * 128), lambda i: (0, 0)),
            pl.BlockSpec((1, hidden), lambda i: (0, 0)),
            pl.BlockSpec((c_out, hidden), lambda i: (0, 0)),
            pl.BlockSpec((1, c_out), lambda i: (0, 0)),
        ],
        out_specs=pl.BlockSpec((_BN, c_out), lambda i: (i, 0)),
        out_shape=jax.ShapeDtypeStruct((n, c_out), jnp.float32),
    )(e3, w1, b1, w2, b2)


def kernel(input_spatial, trunk_out, row_embed, col_embed, W1, b1, W2, b2):
    n = input_spatial.shape[0]
    x2d = input_spatial[:, 9:14, :, :].reshape(n * _NUM_HIST, _HW)
    comb, table = _extract_indices(x2d, row_embed, col_embed)
    emb = _sc_gather(table, comb)
    out = _mlp(emb.reshape(n, _NUM_HIST, 128),
               W1, b1.reshape(1, -1), W2, b2.reshape(1, -1))
    return out[:, :, None, None]


# speedup vs baseline: 1.5305x; 1.5305x over previous
import functools

import jax
import jax.numpy as jnp
from jax import lax
from jax.experimental import pallas as pl
from jax.experimental.pallas import tpu as pltpu
from jax.experimental.pallas import tpu_sc as plsc

_POS_LEN = 19
_PL1 = _POS_LEN + 1
_NUM_HIST = 5
_HW = 361
_PAD_IDX = _PL1 * _PL1 - 1
_CH0 = 9
_BA = 256
_BN = 512


def _argmax_body(x_ref, re_ref, ce_ref, comb_ref, table_ref):
    i = pl.program_id(0)

    @pl.when(i == 0)
    def _build_table():
        p = lax.broadcasted_iota(jnp.int32, (_PL1 * _PL1, _PL1), 0)
        e = lax.broadcasted_iota(jnp.int32, (_PL1 * _PL1, _PL1), 1)
        pr = (p * 410) >> 13
        oh_r = (pr == e).astype(jnp.float32)
        oh_c = (p - pr * _PL1 == e).astype(jnp.float32)
        t_r = jnp.dot(oh_r, re_ref[...], preferred_element_type=jnp.float32)
        t_c = jnp.dot(oh_c, ce_ref[...], preferred_element_type=jnp.float32)
        table_ref[...] = jnp.concatenate([t_r, t_c], axis=-1)

    flat = x_ref[:, pl.ds(_CH0, _NUM_HIST), :]
    s = jnp.sum(flat, axis=-1)
    m = jnp.max(flat, axis=-1)
    iota = lax.broadcasted_iota(jnp.int32, flat.shape, 2).astype(jnp.float32)
    idxf = jnp.min(jnp.where(flat == m[..., None], iota, jnp.float32(512.0)),
                   axis=-1)
    idx = idxf.astype(jnp.int32)
    has = s > 0.5
    rows = (idx * 27) >> 9
    cols = idx - rows * 19
    comb_ref[...] = jnp.where(has, rows * _PL1 + cols, _PAD_IDX)


def _extract_indices(x3, row_embed, col_embed):
    n = x3.shape[0]
    nc = x3.shape[1]
    return pl.pallas_call(
        _argmax_body,
        grid=(n // _BA,),
        in_specs=[
            pl.BlockSpec((_BA, nc, _HW), lambda i: (i, 0, 0)),
            pl.BlockSpec((_PL1, 64), lambda i: (0, 0)),
            pl.BlockSpec((_PL1, 64), lambda i: (0, 0)),
        ],
        out_specs=[
            pl.BlockSpec((_BA, _NUM_HIST), lambda i: (i, 0)),
            pl.BlockSpec((_PL1 * _PL1, 128), lambda i: (0, 0)),
        ],
        out_shape=[
            jax.ShapeDtypeStruct((n, _NUM_HIST), jnp.int32),
            jax.ShapeDtypeStruct((_PL1 * _PL1, 128), jnp.float32),
        ],
    )(x3, row_embed, col_embed)


def _sc_gather(table, idx):
    info = plsc.get_sparse_core_info()
    nw = info.num_cores * info.num_subcores
    n = idx.shape[0]
    rpw = n // nw
    cpw = rpw // 128
    d = table.shape[1]
    mesh = plsc.VectorSubcoreMesh(core_axis_name="c", subcore_axis_name="s")

    @functools.partial(
        pl.kernel,
        mesh=mesh,
        out_type=jax.ShapeDtypeStruct((n, d), jnp.float32),
        scratch_types=[
            pltpu.VMEM((rpw,), jnp.int32),
            pltpu.VMEM((rpw, d), jnp.float32),
            pltpu.SemaphoreType.DMA,
            pltpu.SemaphoreType.DMA,
        ],
    )
    def gather_kernel(table_hbm, idx_hbm, out_hbm, idx_v, rows_v, gsem, wsem):
        wid = lax.axis_index("s") * info.num_cores + lax.axis_index("c")
        base = wid * rpw
        pltpu.sync_copy(idx_hbm.at[pl.ds(base, rpw)], idx_v)
        gathers = [
            pltpu.async_copy(table_hbm.at[idx_v.at[pl.ds(j * 128, 128)]],
                             rows_v.at[pl.ds(j * 128, 128)], gsem)
            for j in range(cpw)
        ]
        writes = []
        for j in range(cpw):
            gathers[j].wait()
            writes.append(pltpu.async_copy(
                rows_v.at[pl.ds(j * 128, 128)],
                out_hbm.at[pl.ds(base + j * 128, 128)], wsem))
        for w in writes:
            w.wait()

    return gather_kernel(table, idx)


def _mlp_body(e_ref, w1_ref, b1_ref, w2_ref, b2_ref, o_ref):
    acc = None
    for k in range(_NUM_HIST):
        part = lax.dot_general(
            e_ref[:, k, :], w1_ref[:, pl.ds(k * 128, 128)],
            (((1,), (1,)), ((), ())), preferred_element_type=jnp.float32)
        acc = part if acc is None else acc + part
    h = jnp.maximum(acc + b1_ref[...], 0.0)
    o_ref[...] = lax.dot_general(
        h, w2_ref[...], (((1,), (1,)), ((), ())),
        preferred_element_type=jnp.float32) + b2_ref[...]


def _mlp(e3, w1, b1, w2, b2):
    n = e3.shape[0]
    hidden = w1.shape[0]
    c_out = w2.shape[0]
    return pl.pallas_call(
        _mlp_body,
        grid=(n // _BN,),
        in_specs=[
            pl.BlockSpec((_BN, _NUM_HIST, 128), lambda i: (i, 0, 0)),
            pl.BlockSpec((hidden, _NUM_HIST * 128), lambda i: (0, 0)),
            pl.BlockSpec((1, hidden), lambda i: (0, 0)),
            pl.BlockSpec((c_out, hidden), lambda i: (0, 0)),
            pl.BlockSpec((1, c_out), lambda i: (0, 0)),
        ],
        out_specs=pl.BlockSpec((_BN, c_out), lambda i: (i, 0)),
        out_shape=jax.ShapeDtypeStruct((n, c_out), jnp.float32),
    )(e3, w1, b1, w2, b2)


def kernel(input_spatial, trunk_out, row_embed, col_embed, W1, b1, W2, b2):
    n = input_spatial.shape[0]
    x3 = input_spatial.reshape(n, input_spatial.shape[1], _HW)
    comb2, table = _extract_indices(x3, row_embed, col_embed)
    comb = comb2.reshape(-1)
    emb = _sc_gather(table, comb)
    out = _mlp(emb.reshape(n, _NUM_HIST, 128),
               W1, b1.reshape(1, -1), W2, b2.reshape(1, -1))
    return out[:, :, None, None]
